# Initial kernel scaffold; baseline (speedup 1.0000x reference)
import jax
import jax.numpy as jnp
from jax.experimental import pallas as pl


def kernel(x, person_id, pose, plane_xy, plane_xz, plane_yz):
    raise NotImplementedError("write your pallas kernel here")


# speedup vs baseline: 3851.3468x; 3851.3468x over previous
import jax
import jax.numpy as jnp
from jax import lax
from jax.experimental import pallas as pl
from jax.experimental.pallas import tpu as pltpu
from jax.experimental.pallas import tpu_sc as plsc

_NPTS = 1048576
_F = 64
_RES = 128
_NC = 2
_NS = 16
_NW = _NC * _NS
_C = 64


def _tri_body(xt_hbm, t4_hbm, out_hbm, xb, idxb, fracb, rows, outb, gsem):
    wid = lax.axis_index("s") * _NC + lax.axis_index("c")
    ppt = _NPTS // _NW
    nchunk = ppt // _C
    base0 = wid * ppt

    def stage(k, b):
        base = base0 + k * _C
        for r in range(3):
            pltpu.sync_copy(xt_hbm.at[r, pl.ds(base, _C)], xb.at[b, r])
        for g in range(_C // 16):
            s = pl.ds(g * 16, 16)
            x0 = xb[b, 0, s]
            x1 = xb[b, 1, s]
            x2 = xb[b, 2, s]
            for j, (a, c) in enumerate(((x0, x1), (x0, x2), (x1, x2))):
                ixf = a * ((_RES - 1) * 0.5) + ((_RES - 1) * 0.5)
                iyf = c * ((_RES - 1) * 0.5) + ((_RES - 1) * 0.5)
                ixw = jnp.minimum(jnp.maximum(ixf.astype(jnp.int32), 0), _RES - 2)
                iyw = jnp.minimum(jnp.maximum(iyf.astype(jnp.int32), 0), _RES - 2)
                fracb[b, 2 * j, s] = ixf - ixw.astype(jnp.float32)
                fracb[b, 2 * j + 1, s] = iyf - iyw.astype(jnp.float32)
                idxb[b, j, s] = (j * (_RES * _RES)) + iyw * _RES + ixw
        for j in range(3):
            pltpu.async_copy(t4_hbm.at[idxb.at[b, j]], rows.at[b, j], gsem.at[b])

    def compute(k, b):
        base = base0 + k * _C
        for j in range(3):
            pltpu.make_async_copy(
                t4_hbm.at[idxb.at[b, j]], rows.at[b, j], gsem.at[b]
            ).wait()

        @pl.loop(0, _C // 16)
        def _grp(g):
            pbase = pl.multiple_of(g * 16, 16)
            fv = [fracb[b, r, pl.ds(pbase, 16)] for r in range(6)]
            for i in range(16):
                p = pbase + i
                accs = [None] * 4
                for j in range(3):
                    fx = fv[2 * j][i]
                    fy = fv[2 * j + 1][i]
                    for q in range(4):
                        nw = rows[b, j, p, pl.ds(q * 16, 16)]
                        ne = rows[b, j, p, pl.ds(_F + q * 16, 16)]
                        sw = rows[b, j, p, pl.ds(2 * _F + q * 16, 16)]
                        se = rows[b, j, p, pl.ds(3 * _F + q * 16, 16)]
                        top = nw + fx * (ne - nw)
                        bot = sw + fx * (se - sw)
                        v = top + fy * (bot - top)
                        accs[q] = v if j == 0 else accs[q] + v
                for q in range(4):
                    outb[b, p, pl.ds(q * 16, 16)] = accs[q] * (1.0 / 3.0)

        pltpu.sync_copy(outb.at[b], out_hbm.at[pl.ds(base, _C)])

    stage(0, 0)

    @pl.loop(0, nchunk, step=2)
    def _outer(k):
        stage(k + 1, 1)
        compute(k, 0)

        @pl.when(k + 2 < nchunk)
        def _():
            stage(k + 2, 0)

        compute(k + 1, 1)


def _run(xt, t4):
    mesh = plsc.VectorSubcoreMesh(core_axis_name="c", subcore_axis_name="s")
    f = pl.kernel(
        _tri_body,
        out_type=jax.ShapeDtypeStruct((_NPTS, _F), jnp.float32),
        mesh=mesh,
        scratch_types=[
            pltpu.VMEM((2, 3, _C), jnp.float32),
            pltpu.VMEM((2, 3, _C), jnp.int32),
            pltpu.VMEM((2, 6, _C), jnp.float32),
            pltpu.VMEM((2, 3, _C, 4 * _F), jnp.float32),
            pltpu.VMEM((2, _C, _F), jnp.float32),
            pltpu.SemaphoreType.DMA((2,)),
        ],
    )
    return f(xt, t4)


def kernel(x, person_id, pose, plane_xy, plane_xz, plane_yz):
    g = jnp.concatenate([plane_xy, plane_xz, plane_yz], axis=0)
    g = g.transpose(0, 2, 3, 1)
    right = jnp.concatenate([g[:, :, 1:], g[:, :, -1:]], axis=2)
    down = jnp.concatenate([g[:, 1:], g[:, -1:]], axis=1)
    dright = jnp.concatenate([down[:, :, 1:], down[:, :, -1:]], axis=2)
    t4 = jnp.concatenate([g, right, down, dright], axis=-1)
    t4 = t4.reshape(3 * _RES * _RES, 4 * _F)
    return _run(x.T, t4)
